# Initial kernel scaffold; baseline (speedup 1.0000x reference)
import jax
import jax.numpy as jnp
from jax.experimental import pallas as pl


def kernel(x, ini_GNN_temp, x_branch, params, sample_idx, edge_index, edge_attr):
    raise NotImplementedError("write your pallas kernel here")


# speedup vs baseline: 1.6157x; 1.6157x over previous
import jax
import jax.numpy as jnp
from jax import lax
from jax.experimental import pallas as pl
from jax.experimental.pallas import tpu as pltpu
from jax.experimental.pallas import tpu_sc as plsc

N = 50000
E = 800000

ROW_BLK = 2048
EDGE_BLK = 2048

NC, NS = 2, 16
NPU = 51200
NPT = NPU // NS
EPAD = 802816
EROWS = EPAD // 128
P2CP = EROWS // NS
P2CW = EROWS // (NC * NS)


def _leaky(x, s):
    return jnp.where(x > 0, x, s * x)


def _sc_mesh():
    return plsc.VectorSubcoreMesh(core_axis_name="c", subcore_axis_name="s",
                                  num_cores=NC, num_subcores=NS)


def _temp_body(xb_ref, w1_ref, b1_ref, w2_ref, b2_ref, g_ref, bb_ref, out_ref):
    xb = xb_ref[:, 6:32]
    h = jnp.dot(xb, w1_ref[...], preferred_element_type=jnp.float32) + b1_ref[...]
    h = _leaky(h, 0.01)
    h = jnp.dot(h, w2_ref[...], preferred_element_type=jnp.float32) + b2_ref[...]
    m = jnp.mean(h, axis=-1, keepdims=True)
    v = jnp.mean((h - m) ** 2, axis=-1, keepdims=True)
    h = (h - m) / jnp.sqrt(v + 1e-5) * g_ref[...] + bb_ref[...]
    out_ref[...] = _leaky(h, 0.01)


def _temp_branch(x_branch, tp):
    S = x_branch.shape[0]
    return pl.pallas_call(
        _temp_body,
        out_shape=jax.ShapeDtypeStruct((S, 64), jnp.float32),
    )(x_branch, tp['W1'], tp['b1'].reshape(1, 96), tp['W2'],
      tp['b2'].reshape(1, 64), tp['ln_g'].reshape(1, 64), tp['ln_b'].reshape(1, 64))


def _hw0_body(h_ref, w_ref, b_ref, out_ref):
    out = jnp.dot(h_ref[...], w_ref[...], preferred_element_type=jnp.float32) + b_ref[...]
    out_ref[...] = jnp.pad(out, ((0, 0), (0, 64)))


def _hw_first(h0p, Wp, bn):
    grid = NPU // ROW_BLK
    K = h0p.shape[1]
    return pl.pallas_call(
        _hw0_body,
        grid=(grid,),
        in_specs=[
            pl.BlockSpec((ROW_BLK, K), lambda i: (i, 0)),
            pl.BlockSpec((K, 64), lambda i: (0, 0)),
            pl.BlockSpec((1, 64), lambda i: (0, 0)),
        ],
        out_specs=pl.BlockSpec((ROW_BLK, 128), lambda i: (i, 0)),
        out_shape=jax.ShapeDtypeStruct((NPU, 128), jnp.float32),
    )(h0p, Wp, bn.reshape(1, 64))


def _u_specs():
    return [
        pl.BlockSpec((1, ROW_BLK, 16), lambda i: (0, i, 0)),
        pl.BlockSpec((1, ROW_BLK, 16), lambda i: (1, i, 0)),
        pl.BlockSpec((1, ROW_BLK, 16), lambda i: (0, i, 0)),
        pl.BlockSpec((1, ROW_BLK, 16), lambda i: (1, i, 0)),
    ]


def _cat_u(ua0_ref, ua1_ref, ub0_ref, ub1_ref):
    return jnp.concatenate(
        [ua0_ref[0], ub0_ref[0], ua1_ref[0], ub1_ref[0]], axis=1)


def _hwn_body(ua0_ref, ua1_ref, ub0_ref, ub1_ref, rcp_ref, am_ref, rsw_ref,
              gb_ref, w_ref, b_ref, out_ref):
    h = _cat_u(ua0_ref, ua1_ref, ub0_ref, ub1_ref) * rcp_ref[...]
    h = (h - am_ref[...]) * rsw_ref[...] + gb_ref[...]
    h = _leaky(h, 0.1)
    out = jnp.dot(h, w_ref[...], preferred_element_type=jnp.float32) + b_ref[...]
    out_ref[...] = jnp.pad(out, ((0, 0), (0, 64)))


def _hw_next(uA, uB, rcp, am, rsw, gb, Wn, bn):
    grid = NPU // ROW_BLK
    return pl.pallas_call(
        _hwn_body,
        grid=(grid,),
        in_specs=_u_specs() + [
            pl.BlockSpec((ROW_BLK, 1), lambda i: (i, 0)),
            pl.BlockSpec((1, 64), lambda i: (0, 0)),
            pl.BlockSpec((1, 64), lambda i: (0, 0)),
            pl.BlockSpec((1, 64), lambda i: (0, 0)),
            pl.BlockSpec((64, 64), lambda i: (0, 0)),
            pl.BlockSpec((1, 64), lambda i: (0, 0)),
        ],
        out_specs=pl.BlockSpec((ROW_BLK, 128), lambda i: (i, 0)),
        out_shape=jax.ShapeDtypeStruct((NPU, 128), jnp.float32),
    )(uA, uA, uB, uB, rcp, am.reshape(1, 64), rsw.reshape(1, 64),
      gb.reshape(1, 64), Wn, bn.reshape(1, 64))


def _edge_body(g_ref, ea_ref, we_ref, be_ref, attn_ref,
               v0_ref, v1_ref, v2_ref, v3_ref, ex_ref):
    g = g_ref[:, :64]
    ea = ea_ref[...]
    we = we_ref[...]
    e = (ea[:, 0:1] * we[0:1, :] + ea[:, 1:2] * we[1:2, :]
         + ea[:, 2:3] * we[2:3, :] + be_ref[...])
    z = g * jax.nn.sigmoid(e)
    alpha = jnp.sum(z * attn_ref[...], axis=-1)
    alpha = _leaky(alpha, 0.1)
    ex = jnp.exp(alpha)
    val = z * ex[:, None]
    v0_ref[...] = val[:, 0:16]
    v1_ref[...] = val[:, 16:32]
    v2_ref[...] = val[:, 32:48]
    v3_ref[...] = val[:, 48:64]
    col = lax.broadcasted_iota(jnp.int32, (EDGE_BLK, 16), 1)
    ex_ref[...] = jnp.where(col == 0, ex[:, None], 0.0)


def _tc_edge(g128, ea_p, We, be, attn):
    grid = EPAD // EDGE_BLK
    return pl.pallas_call(
        _edge_body,
        grid=(grid,),
        in_specs=[
            pl.BlockSpec((EDGE_BLK, 128), lambda i: (i, 0)),
            pl.BlockSpec((EDGE_BLK, 3), lambda i: (i, 0)),
            pl.BlockSpec((3, 64), lambda i: (0, 0)),
            pl.BlockSpec((1, 64), lambda i: (0, 0)),
            pl.BlockSpec((1, 64), lambda i: (0, 0)),
        ],
        out_specs=[
            pl.BlockSpec((EDGE_BLK, 16), lambda i: (i, 0)),
            pl.BlockSpec((EDGE_BLK, 16), lambda i: (i, 0)),
            pl.BlockSpec((EDGE_BLK, 16), lambda i: (i, 0)),
            pl.BlockSpec((EDGE_BLK, 16), lambda i: (i, 0)),
            pl.BlockSpec((EDGE_BLK, 16), lambda i: (i, 0)),
        ],
        out_shape=[
            jax.ShapeDtypeStruct((EPAD, 16), jnp.float32),
            jax.ShapeDtypeStruct((EPAD, 16), jnp.float32),
            jax.ShapeDtypeStruct((EPAD, 16), jnp.float32),
            jax.ShapeDtypeStruct((EPAD, 16), jnp.float32),
            jax.ShapeDtypeStruct((EPAD, 16), jnp.float32),
        ],
    )(g128, ea_p, We, be.reshape(1, 64), attn.reshape(1, 64))


def _gather_body(hw_h, src_h, g_h, idx_v, g_v, sem):
    c = lax.axis_index("c")
    s = lax.axis_index("s")
    w = s * NC + c

    def chunk(k, _):
        r = w * P2CW + k
        pltpu.sync_copy(src_h.at[pl.ds(r * 128, 128)], idx_v)
        pltpu.async_copy(hw_h.at[idx_v], g_v, sem).wait()
        pltpu.sync_copy(g_v, g_h.at[pl.ds(r * 128, 128)])
        return 0

    lax.fori_loop(0, P2CW, chunk, 0)


def _sc_gather(hw128, src1):
    f = pl.kernel(
        _gather_body,
        out_type=jax.ShapeDtypeStruct((EPAD, 128), jnp.float32),
        mesh=_sc_mesh(),
        scratch_types=[
            pltpu.VMEM((128,), jnp.int32),
            pltpu.VMEM((128, 128), jnp.float32),
            pltpu.SemaphoreType.DMA,
        ],
    )
    return f(hw128, src1)


def _stats_body(ua0_ref, ua1_ref, ub0_ref, ub1_ref, rcp_ref, s1_ref, s2_ref):
    h = _cat_u(ua0_ref, ua1_ref, ub0_ref, ub1_ref) * rcp_ref[...]
    gid = pl.program_id(0) * ROW_BLK + lax.broadcasted_iota(
        jnp.int32, (ROW_BLK, 1), 0)
    h = jnp.where(gid < N, h, 0.0)

    @pl.when(pl.program_id(0) == 0)
    def _():
        s1_ref[...] = jnp.zeros_like(s1_ref)
        s2_ref[...] = jnp.zeros_like(s2_ref)

    s1_ref[...] += jnp.sum(h, axis=0, keepdims=True)
    s2_ref[...] += jnp.sum(h * h, axis=0, keepdims=True)


def _stats(uA, uB, rcp):
    grid = NPU // ROW_BLK
    return pl.pallas_call(
        _stats_body,
        grid=(grid,),
        in_specs=_u_specs() + [
            pl.BlockSpec((ROW_BLK, 1), lambda i: (i, 0)),
        ],
        out_specs=[
            pl.BlockSpec((1, 64), lambda i: (0, 0)),
            pl.BlockSpec((1, 64), lambda i: (0, 0)),
        ],
        out_shape=[
            jax.ShapeDtypeStruct((1, 64), jnp.float32),
            jax.ShapeDtypeStruct((1, 64), jnp.float32),
        ],
    )(uA, uA, uB, uB, rcp)


def _res_head(uA, uB, rcp, am, rsw, gb, rp, ini_p):
    grid = NPU // ROW_BLK

    def body(ua0_ref, ua1_ref, ub0_ref, ub1_ref, rcp_ref, am_ref, rsw_ref,
             gb_ref, w1_ref, b1_ref, w2_ref, b2_ref, ini_ref, out_ref):
        h = _cat_u(ua0_ref, ua1_ref, ub0_ref, ub1_ref) * rcp_ref[...]
        h = (h - am_ref[...]) * rsw_ref[...] + gb_ref[...]
        h = _leaky(h, 0.1)
        dz = jnp.dot(h, w1_ref[...], preferred_element_type=jnp.float32) + b1_ref[...]
        dz = _leaky(dz, 0.1)
        z2 = jnp.sum(dz * w2_ref[...], axis=-1, keepdims=True) + b2_ref[0, 0]
        out_ref[...] = ini_ref[...] + jnp.tanh(z2)

    return pl.pallas_call(
        body,
        grid=(grid,),
        in_specs=_u_specs() + [
            pl.BlockSpec((ROW_BLK, 1), lambda i: (i, 0)),
            pl.BlockSpec((1, 64), lambda i: (0, 0)),
            pl.BlockSpec((1, 64), lambda i: (0, 0)),
            pl.BlockSpec((1, 64), lambda i: (0, 0)),
            pl.BlockSpec((64, 32), lambda i: (0, 0)),
            pl.BlockSpec((1, 32), lambda i: (0, 0)),
            pl.BlockSpec((1, 32), lambda i: (0, 0)),
            pl.BlockSpec((1, 1), lambda i: (0, 0)),
            pl.BlockSpec((ROW_BLK, 1), lambda i: (i, 0)),
        ],
        out_specs=pl.BlockSpec((ROW_BLK, 1), lambda i: (i, 0)),
        out_shape=jax.ShapeDtypeStruct((NPU, 1), jnp.float32),
    )(uA, uA, uB, uB, rcp, am.reshape(1, 64), rsw.reshape(1, 64),
      gb.reshape(1, 64), rp['W1'], rp['b1'].reshape(1, 32),
      rp['W2'].reshape(1, 32), rp['b2'].reshape(1, 1), ini_p)


def kernel(x, ini_GNN_temp, x_branch, params, sample_idx, edge_index, edge_attr):
    src1 = jnp.pad(edge_index[0], (0, EPAD - E))
    dst1 = jnp.pad(edge_index[1], (0, EPAD - E), constant_values=N)
    ea_p = jnp.pad(edge_attr, ((0, EPAD - E), (0, 0)))

    enc = _temp_branch(x_branch, params['temp'])
    temp_features = jnp.zeros((N, 64), jnp.float32).at[sample_idx].set(enc)
    keep = jnp.array([0, 1, 2, 4, 5, 6])
    h0 = jnp.concatenate([x[:, keep], ini_GNN_temp, temp_features], axis=1)
    h0p = jnp.pad(h0, ((0, NPU - N), (0, 9)))
    ini_p = jnp.pad(ini_GNN_temp, ((0, NPU - N), (0, 0)))

    nstats = None
    uA = uB = rcp = None
    for li, lp in enumerate(params['layers']):
        if li == 0:
            Wp = jnp.pad(lp['Wn'], ((0, 9), (0, 0)))
            hw128 = _hw_first(h0p, Wp, lp['bn'])
        else:
            am, rsw, gb = nstats
            hw128 = _hw_next(uA, uB, rcp, am, rsw, gb, lp['Wn'], lp['bn'])

        g128 = _sc_gather(hw128, src1)
        v0, v1, v2, v3, ex16 = _tc_edge(g128, ea_p, lp['We'], lp['be'],
                                        lp['attn'][0])
        uA = jnp.stack([jax.ops.segment_sum(v0, dst1, num_segments=NPU),
                        jax.ops.segment_sum(v2, dst1, num_segments=NPU)])
        uB = jnp.stack([jax.ops.segment_sum(v1, dst1, num_segments=NPU),
                        jax.ops.segment_sum(v3, dst1, num_segments=NPU)])
        uC = jnp.stack([jax.ops.segment_sum(ex16, dst1, num_segments=NPU),
                        jnp.zeros((NPU, 16), jnp.float32)])
        den = uC[0, :, 0] + uC[1, :, 0]
        rcp = (1.0 / (den + 1e-16)).reshape(NPU, 1)

        s1, s2 = _stats(uA, uB, rcp)
        m = s1[0] / N
        var = s2[0] / N - (2.0 * lp['gn_a'] - lp['gn_a'] ** 2) * m * m
        rs = 1.0 / jnp.sqrt(var + 1e-5)
        nstats = (lp['gn_a'] * m, rs * lp['gn_w'], lp['gn_b'])

    am, rsw, gb = nstats
    out = _res_head(uA, uB, rcp, am, rsw, gb, params['res'], ini_p)
    return out[:N]
